# baseline (speedup 1.0000x reference)
import functools

import jax
import jax.numpy as jnp
from jax import lax
from jax.experimental import pallas as pl
from jax.experimental.pallas import tpu as pltpu
from jax.experimental.pallas import tpu_sc as plsc

B, L, D = 16384, 50, 128
VOCAB = 24
MAX_CHARGE = 10
ROWS = L * VOCAB * MAX_CHARGE
CAT = L + VOCAB + MAX_CHARGE

NC, NS = 2, 16
NW = NC * NS
TOK = B * L
B_PER_W = B // NW
TOK_PER_W = B_PER_W * L
PIECE_B = 4
PTOK = PIECE_B * L
GCHUNKS = ((0, 128), (128, 72))
NPIECE = B_PER_W // PIECE_B


def _pos_enc():
    pos = jnp.arange(L, dtype=jnp.float32)[:, None]
    i = jnp.arange(D // 2, dtype=jnp.float32)[None, :]
    angle = pos / jnp.power(10000.0, (2.0 * i) / D)
    return jnp.stack([jnp.sin(angle), jnp.cos(angle)], axis=-1).reshape(L, D)


def _tc_prep(cat_ref, tok_ref, ch_ref, mega_ref, idx_ref):
    r = lax.broadcasted_iota(jnp.int32, (ROWS, CAT), 0)
    col = lax.broadcasted_iota(jnp.int32, (ROWS, CAT), 1)
    l = r // (VOCAB * MAX_CHARGE)
    v = (r // MAX_CHARGE) % VOCAB
    c = r % MAX_CHARGE
    oh = ((col == l) | (col == L + v) | (col == L + VOCAB + c)).astype(jnp.float32)
    mega_ref[...] = jnp.dot(oh, cat_ref[...], preferred_element_type=jnp.float32)
    li = lax.broadcasted_iota(jnp.int32, (B, L), 1)
    idx_ref[...] = li * (VOCAB * MAX_CHARGE) + tok_ref[...] * MAX_CHARGE + ch_ref[...]


def _sc_gather(mega_hbm, idx_hbm, out_hbm, idx_v, buf0, buf1, g0, g1, s0, s1):
    wid = lax.axis_index("s") * NC + lax.axis_index("c")
    b_base = wid * B_PER_W
    pltpu.sync_copy(idx_hbm.at[pl.ds(wid * TOK_PER_W, TOK_PER_W)], idx_v)

    def piece(i, p, buf, gsem, ssem):
        @pl.when(i > 0)
        def _():
            for _ in range(PIECE_B):
                pltpu.make_async_copy(buf.at[pl.ds(0, L)], out_hbm.at[0], ssem).wait()

        t0 = p * PTOK
        gs = [
            pltpu.async_copy(
                mega_hbm.at[idx_v.at[pl.ds(t0 + off, n)]], buf.at[pl.ds(off, n)], gsem
            )
            for off, n in GCHUNKS
        ]
        for g in gs:
            g.wait()
        b0 = b_base + p * PIECE_B
        for j in range(PIECE_B):
            pltpu.async_copy(buf.at[pl.ds(j * L, L)], out_hbm.at[b0 + j], ssem)

    def body(i, _):
        piece(i, 2 * i, buf0, g0, s0)
        piece(i, 2 * i + 1, buf1, g1, s1)
        return 0

    lax.fori_loop(0, NPIECE // 2, body, 0)
    for buf, ssem in ((buf0, s0), (buf1, s1)):
        for _ in range(PIECE_B):
            pltpu.make_async_copy(buf.at[pl.ds(0, L)], out_hbm.at[0], ssem).wait()


def kernel(tokens, charges, aa_table, charge_table):
    cat = jnp.concatenate([_pos_enc(), aa_table, charge_table], axis=0)
    mega, idx = pl.pallas_call(
        _tc_prep,
        out_shape=[
            jax.ShapeDtypeStruct((ROWS, D), jnp.float32),
            jax.ShapeDtypeStruct((B, L), jnp.int32),
        ],
    )(cat, tokens, charges.reshape(B, 1))
    idx1d = idx.reshape(TOK)

    sc = functools.partial(
        pl.kernel,
        out_type=jax.ShapeDtypeStruct((B, L, D), jnp.float32),
        mesh=plsc.VectorSubcoreMesh(core_axis_name="c", subcore_axis_name="s"),
        compiler_params=pltpu.CompilerParams(use_tc_tiling_on_sc=True),
        scratch_types=[
            pltpu.VMEM((TOK_PER_W,), jnp.int32),
            pltpu.VMEM((PTOK, D), jnp.float32),
            pltpu.VMEM((PTOK, D), jnp.float32),
            pltpu.SemaphoreType.DMA,
            pltpu.SemaphoreType.DMA,
            pltpu.SemaphoreType.DMA,
            pltpu.SemaphoreType.DMA,
        ],
    )(_sc_gather)
    return sc(mega, idx1d)


# speedup vs baseline: 1.8497x; 1.8497x over previous
import functools

import jax
import jax.numpy as jnp
from jax import lax
from jax.experimental import pallas as pl
from jax.experimental.pallas import tpu as pltpu
from jax.experimental.pallas import tpu_sc as plsc

B, L, D = 16384, 50, 128
VOCAB = 24
MAX_CHARGE = 10
ROWS = L * VOCAB * MAX_CHARGE
CAT = L + VOCAB + MAX_CHARGE

NC, NS = 2, 16
NW = NC * NS
TOK = B * L
TOK_PER_W = TOK // NW
PTOK = 256
NG = PTOK // 128
NPIECE = TOK_PER_W // PTOK


def _pos_enc():
    pos = jnp.arange(L, dtype=jnp.float32)[:, None]
    i = jnp.arange(D // 2, dtype=jnp.float32)[None, :]
    angle = pos / jnp.power(10000.0, (2.0 * i) / D)
    return jnp.stack([jnp.sin(angle), jnp.cos(angle)], axis=-1).reshape(L, D)


def _tc_prep(cat_ref, tokT_ref, ch_ref, mega_ref, idxT_ref):
    r = lax.broadcasted_iota(jnp.int32, (ROWS, CAT), 0)
    col = lax.broadcasted_iota(jnp.int32, (ROWS, CAT), 1)
    l = r // (VOCAB * MAX_CHARGE)
    v = (r // MAX_CHARGE) % VOCAB
    c = r % MAX_CHARGE
    oh = ((col == l) | (col == L + v) | (col == L + VOCAB + c)).astype(jnp.float32)
    mega_ref[...] = jnp.dot(oh, cat_ref[...], preferred_element_type=jnp.float32)
    li = lax.broadcasted_iota(jnp.int32, (TOK // D, D), 0) // (B // D)
    ch_big = jnp.broadcast_to(ch_ref[...][None], (L, B // D, D)).reshape(TOK // D, D)
    idxT_ref[...] = (
        li * (VOCAB * MAX_CHARGE) + tokT_ref[...] * MAX_CHARGE + ch_big
    )


def _sc_gather(mega_hbm, idx_hbm, out_hbm, idx_v, buf0, buf1, g0, g1, s0, s1):
    wid = lax.axis_index("s") * NC + lax.axis_index("c")
    base = wid * TOK_PER_W
    pltpu.sync_copy(idx_hbm.at[pl.ds(base, TOK_PER_W)], idx_v)

    def piece(i, p, buf, gsem, ssem):
        @pl.when(i > 0)
        def _():
            pltpu.make_async_copy(buf, out_hbm.at[pl.ds(base, PTOK)], ssem).wait()

        t0 = p * PTOK
        gs = [
            pltpu.async_copy(
                mega_hbm.at[idx_v.at[pl.ds(t0 + g * 128, 128)]],
                buf.at[pl.ds(g * 128, 128)],
                gsem,
            )
            for g in range(NG)
        ]
        for g in gs:
            g.wait()
        pltpu.async_copy(buf, out_hbm.at[pl.ds(base + t0, PTOK)], ssem)

    def body(i, _):
        piece(i, 2 * i, buf0, g0, s0)
        piece(i, 2 * i + 1, buf1, g1, s1)
        return 0

    lax.fori_loop(0, NPIECE // 2, body, 0)
    for buf, ssem in ((buf0, s0), (buf1, s1)):
        pltpu.make_async_copy(buf, out_hbm.at[pl.ds(base, PTOK)], ssem).wait()


def kernel(tokens, charges, aa_table, charge_table):
    cat = jnp.concatenate([_pos_enc(), aa_table, charge_table], axis=0)
    tokT = tokens.T.reshape(TOK // D, D)
    mega, idxT = pl.pallas_call(
        _tc_prep,
        out_shape=[
            jax.ShapeDtypeStruct((ROWS, D), jnp.float32),
            jax.ShapeDtypeStruct((TOK // D, D), jnp.int32),
        ],
    )(cat, tokT, charges.reshape(B // D, D))
    idx1d = idxT.reshape(TOK)

    sc = functools.partial(
        pl.kernel,
        out_type=jax.ShapeDtypeStruct((TOK, D), jnp.float32),
        mesh=plsc.VectorSubcoreMesh(
            core_axis_name="c", subcore_axis_name="s", num_cores=NC, num_subcores=NS
        ),
        scratch_types=[
            pltpu.VMEM((TOK_PER_W,), jnp.int32),
            pltpu.VMEM((PTOK, D), jnp.float32),
            pltpu.VMEM((PTOK, D), jnp.float32),
            pltpu.SemaphoreType.DMA,
            pltpu.SemaphoreType.DMA,
            pltpu.SemaphoreType.DMA,
            pltpu.SemaphoreType.DMA,
        ],
    )(_sc_gather)
    out_lmajor = sc(mega, idx1d)
    return out_lmajor.reshape(L, B, D).transpose(1, 0, 2)


# speedup vs baseline: 3.0560x; 1.6522x over previous
import functools

import jax
import jax.numpy as jnp
from jax import lax
from jax.experimental import pallas as pl
from jax.experimental.pallas import tpu as pltpu
from jax.experimental.pallas import tpu_sc as plsc

B, L, D = 16384, 50, 128
VOCAB = 24
MAX_CHARGE = 10
ROWS = L * VOCAB * MAX_CHARGE
CAT = L + VOCAB + MAX_CHARGE

NC, NS = 2, 16
NW = NC * NS
TOK = B * L
TOK_PER_W = TOK // NW
PTOK = 128
NPIECE = TOK_PER_W // PTOK


def _pos_enc():
    pos = jnp.arange(L, dtype=jnp.float32)[:, None]
    i = jnp.arange(D // 2, dtype=jnp.float32)[None, :]
    angle = pos / jnp.power(10000.0, (2.0 * i) / D)
    return jnp.stack([jnp.sin(angle), jnp.cos(angle)], axis=-1).reshape(L, D)


def _tc_prep(cat_ref, tokT_ref, ch_ref, mega_ref, idxT_ref):
    r = lax.broadcasted_iota(jnp.int32, (ROWS, CAT), 0)
    col = lax.broadcasted_iota(jnp.int32, (ROWS, CAT), 1)
    l = r // (VOCAB * MAX_CHARGE)
    v = (r // MAX_CHARGE) % VOCAB
    c = r % MAX_CHARGE
    oh = ((col == l) | (col == L + v) | (col == L + VOCAB + c)).astype(jnp.float32)
    mega_ref[...] = jnp.dot(oh, cat_ref[...], preferred_element_type=jnp.float32)
    li = lax.broadcasted_iota(jnp.int32, (TOK // D, D), 0) // (B // D)
    ch_big = jnp.broadcast_to(ch_ref[...][None], (L, B // D, D)).reshape(TOK // D, D)
    idxT_ref[...] = (
        li * (VOCAB * MAX_CHARGE) + tokT_ref[...] * MAX_CHARGE + ch_big
    )


def _sc_gather(
    mega_hbm, idx_hbm, out_hbm,
    mega_sp, idx0, idx1, buf0, buf1,
    i0, i1, g0, g1, s0, s1,
):
    sid = lax.axis_index("s")
    wid = sid * NC + lax.axis_index("c")
    base = wid * TOK_PER_W

    pltpu.async_copy(idx_hbm.at[pl.ds(base, PTOK)], idx0, i0)
    pltpu.async_copy(idx_hbm.at[pl.ds(base + PTOK, PTOK)], idx1, i1)

    @pl.when(sid == 0)
    def _():
        pltpu.sync_copy(mega_hbm, mega_sp)

    plsc.subcore_barrier()

    def piece(i, p, idxb, buf, isem, gsem, ssem):
        @pl.when(i > 0)
        def _():
            pltpu.make_async_copy(buf, out_hbm.at[pl.ds(base, PTOK)], ssem).wait()

        pltpu.make_async_copy(idx_hbm.at[pl.ds(base, PTOK)], idxb, isem).wait()
        pltpu.async_copy(mega_sp.at[idxb], buf, gsem).wait()
        pltpu.async_copy(buf, out_hbm.at[pl.ds(base + p * PTOK, PTOK)], ssem)

        @pl.when(p + 2 < NPIECE)
        def _():
            pltpu.async_copy(
                idx_hbm.at[pl.ds(base + (p + 2) * PTOK, PTOK)], idxb, isem
            )

    def body(i, _):
        piece(i, 2 * i, idx0, buf0, i0, g0, s0)
        piece(i, 2 * i + 1, idx1, buf1, i1, g1, s1)
        return 0

    lax.fori_loop(0, NPIECE // 2, body, 0)
    for buf, ssem in ((buf0, s0), (buf1, s1)):
        pltpu.make_async_copy(buf, out_hbm.at[pl.ds(base, PTOK)], ssem).wait()


def kernel(tokens, charges, aa_table, charge_table):
    cat = jnp.concatenate([_pos_enc(), aa_table, charge_table], axis=0)
    tokT = tokens.T.reshape(TOK // D, D)
    mega, idxT = pl.pallas_call(
        _tc_prep,
        out_shape=[
            jax.ShapeDtypeStruct((ROWS, D), jnp.float32),
            jax.ShapeDtypeStruct((TOK // D, D), jnp.int32),
        ],
    )(cat, tokT, charges.reshape(B // D, D))
    idx1d = idxT.reshape(TOK)

    sc = functools.partial(
        pl.kernel,
        out_type=jax.ShapeDtypeStruct((TOK, D), jnp.float32),
        mesh=plsc.VectorSubcoreMesh(
            core_axis_name="c", subcore_axis_name="s", num_cores=NC, num_subcores=NS
        ),
        scratch_types=[
            pltpu.VMEM_SHARED((ROWS, D), jnp.float32),
            pltpu.VMEM((PTOK,), jnp.int32),
            pltpu.VMEM((PTOK,), jnp.int32),
            pltpu.VMEM((PTOK, D), jnp.float32),
            pltpu.VMEM((PTOK, D), jnp.float32),
            pltpu.SemaphoreType.DMA,
            pltpu.SemaphoreType.DMA,
            pltpu.SemaphoreType.DMA,
            pltpu.SemaphoreType.DMA,
            pltpu.SemaphoreType.DMA,
            pltpu.SemaphoreType.DMA,
        ],
    )(_sc_gather)
    out_lmajor = sc(mega, idx1d)
    return out_lmajor.reshape(L, B, D).transpose(1, 0, 2)
